# baseline (speedup 1.0000x reference)
import functools

import jax
import jax.numpy as jnp
from jax import lax
from jax.experimental import pallas as pl
from jax.experimental.pallas import tpu as pltpu
from jax.experimental.pallas import tpu_sc as plsc

_NBINS = 16
_NC, _NS, _L = 2, 16, 16
_NW = _NC * _NS
_CH = 16384
_UNROLL = 8
_TSTRIDE = 33


def _pwla_call(m):
    n_chunks = m // (_NW * _CH)
    per_w = m // _NW

    mesh = plsc.VectorSubcoreMesh(
        core_axis_name="c", subcore_axis_name="s",
        num_cores=_NC, num_subcores=_NS)

    @functools.partial(
        pl.kernel,
        out_type=jax.ShapeDtypeStruct((m,), jnp.float32),
        mesh=mesh,
        compiler_params=pltpu.CompilerParams(needs_layout_passes=False),
        scratch_types=[
            pltpu.VMEM((_CH,), jnp.float32),
            pltpu.VMEM((_CH,), jnp.float32),
            pltpu.VMEM((_CH,), jnp.float32),
            pltpu.VMEM((_CH,), jnp.float32),
            pltpu.VMEM((_L, _TSTRIDE), jnp.float32),
            pltpu.VMEM((_L, _TSTRIDE), jnp.float32),
            pltpu.VMEM((_L,), jnp.float32),
            pltpu.VMEM((_L,), jnp.float32),
            pltpu.SemaphoreType.DMA,
            pltpu.SemaphoreType.DMA,
            pltpu.SemaphoreType.DMA,
            pltpu.SemaphoreType.DMA,
        ],
    )
    def k(x_hbm, a_hbm, k_hbm, sc_hbm, sh_hbm, out_hbm,
          xin0, xin1, yout0, yout1, a_v, k_v, sc_v, sh_v,
          isem0, isem1, osem0, osem1):
        wid = lax.axis_index("s") * _NC + lax.axis_index("c")
        base = wid * per_w

        pltpu.sync_copy(a_hbm, a_v)
        pltpu.sync_copy(k_hbm, k_v)
        pltpu.sync_copy(sc_hbm, sc_v)
        pltpu.sync_copy(sh_hbm, sh_v)
        scale = sc_v[...]
        shift = sh_v[...]
        lane = lax.iota(jnp.int32, _L)

        xbufs = (xin0, xin1)
        ybufs = (yout0, yout1)
        isems = (isem0, isem1)
        osems = (osem0, osem1)

        def in_copy(c, b):
            return pltpu.make_async_copy(
                x_hbm.at[pl.ds(base + c * _CH, _CH)], xbufs[b], isems[b])

        def out_copy(c, b):
            return pltpu.make_async_copy(
                ybufs[b], out_hbm.at[pl.ds(base + c * _CH, _CH)], osems[b])

        def compute(b):
            xin_v = xbufs[b]
            yout_v = ybufs[b]

            @plsc.parallel_loop(0, _CH, step=_L, unroll=_UNROLL)
            def vbody(off):
                xv = xin_v[pl.ds(off, _L)]
                t = jnp.minimum(
                    jnp.maximum(xv * scale + shift, 0.0),
                    float(_NBINS + 1))
                i = t.astype(jnp.int32)
                av = plsc.load_gather(a_v, [lane, i])
                kv = plsc.load_gather(k_v, [lane, i])
                yout_v[pl.ds(off, _L)] = av + xv * kv

        in_copy(0, 0).start()
        in_copy(1, 1).start()
        for b in range(2):
            in_copy(b, b).wait()
            compute(b)
            out_copy(b, b).start()
            in_copy(b + 2, b).start()

        def pair(p, carry):
            for b in range(2):
                c = 2 * p + b
                in_copy(c, b).wait()
                out_copy(c - 2, b).wait()
                compute(b)
                out_copy(c, b).start()
                in_copy(c + 2, b).start()
            return carry

        lax.fori_loop(1, n_chunks // 2 - 1, pair, 0)

        for b in range(2):
            c = n_chunks - 2 + b
            in_copy(c, b).wait()
            out_copy(c - 2, b).wait()
            compute(b)
            out_copy(c, b).start()
        for b in range(2):
            out_copy(n_chunks - 2 + b, b).wait()

    return k


def kernel(x, mode, Br, Bl, Kl, Kr, Yidx):
    del mode
    orig_shape = x.shape
    xf = x.reshape(-1)
    m = xf.shape[0]

    f32 = jnp.float32
    Br = Br.astype(f32)
    Bl = Bl.astype(f32)
    inv_d = _NBINS / (Br - Bl)
    d = (Br - Bl) / _NBINS

    j = jnp.arange(_NBINS, dtype=f32)
    k_in = (Yidx[1:] - Yidx[:-1]) * inv_d
    b_j = Bl + j * d
    a_in = Yidx[:-1] - b_j * k_in
    k_full = jnp.concatenate(
        [Kl[None].astype(f32), k_in, Kr[None].astype(f32)])
    a_full = jnp.concatenate(
        [(Yidx[0] - Bl * Kl)[None], a_in, (Yidx[-1] - Br * Kr)[None]])
    a_row = jnp.zeros((_TSTRIDE,), f32).at[: _NBINS + 2].set(a_full)
    k_row = jnp.zeros((_TSTRIDE,), f32).at[: _NBINS + 2].set(k_full)
    a_tab = jnp.tile(a_row[None, :], (_L, 1))
    k_tab = jnp.tile(k_row[None, :], (_L, 1))

    scale_vec = jnp.full((_L,), inv_d, f32)
    shift_vec = jnp.full((_L,), 1.0 - Bl * inv_d, f32)

    out = _pwla_call(m)(xf, a_tab, k_tab, scale_vec, shift_vec)
    return out.reshape(orig_shape)


# speedup vs baseline: 2.0791x; 2.0791x over previous
import functools

import jax
import jax.numpy as jnp
from jax import lax
from jax.experimental import pallas as pl
from jax.experimental.pallas import tpu as pltpu
from jax.experimental.pallas import tpu_sc as plsc

_NBINS = 16
_NC, _NS, _L = 2, 16, 16
_NW = _NC * _NS
_CH = 16384
_UNROLL = 8


def _pwla_call(m):
    n_chunks = m // (_NW * _CH)
    per_w = m // _NW

    mesh = plsc.VectorSubcoreMesh(
        core_axis_name="c", subcore_axis_name="s",
        num_cores=_NC, num_subcores=_NS)

    @functools.partial(
        pl.kernel,
        out_type=jax.ShapeDtypeStruct((m,), jnp.float32),
        mesh=mesh,
        compiler_params=pltpu.CompilerParams(needs_layout_passes=False),
        scratch_types=[
            pltpu.VMEM((_CH,), jnp.float32),
            pltpu.VMEM((_CH,), jnp.float32),
            pltpu.VMEM((_CH,), jnp.float32),
            pltpu.VMEM((_CH,), jnp.float32),
            pltpu.VMEM((32,), jnp.float32),
            pltpu.VMEM((32,), jnp.float32),
            pltpu.VMEM((_L,), jnp.float32),
            pltpu.VMEM((_L,), jnp.float32),
            pltpu.SemaphoreType.DMA,
            pltpu.SemaphoreType.DMA,
            pltpu.SemaphoreType.DMA,
            pltpu.SemaphoreType.DMA,
        ],
    )
    def k(x_hbm, a_hbm, k_hbm, sc_hbm, sh_hbm, out_hbm,
          xin0, xin1, yout0, yout1, a_v, k_v, sc_v, sh_v,
          isem0, isem1, osem0, osem1):
        wid = lax.axis_index("s") * _NC + lax.axis_index("c")
        base = wid * per_w

        pltpu.sync_copy(a_hbm, a_v)
        pltpu.sync_copy(k_hbm, k_v)
        pltpu.sync_copy(sc_hbm, sc_v)
        pltpu.sync_copy(sh_hbm, sh_v)
        scale = sc_v[...]
        shift = sh_v[...]

        xbufs = (xin0, xin1)
        ybufs = (yout0, yout1)
        isems = (isem0, isem1)
        osems = (osem0, osem1)

        def in_copy(c, b):
            return pltpu.make_async_copy(
                x_hbm.at[pl.ds(base + c * _CH, _CH)], xbufs[b], isems[b])

        def out_copy(c, b):
            return pltpu.make_async_copy(
                ybufs[b], out_hbm.at[pl.ds(base + c * _CH, _CH)], osems[b])

        def compute(b):
            xin_v = xbufs[b]
            yout_v = ybufs[b]

            @plsc.parallel_loop(0, _CH, step=_L, unroll=_UNROLL)
            def vbody(off):
                xv = xin_v[pl.ds(off, _L)]
                t = jnp.minimum(
                    jnp.maximum(xv * scale + shift, 0.0),
                    float(_NBINS + 1))
                i = t.astype(jnp.int32)
                av = plsc.load_gather(a_v, [i])
                kv = plsc.load_gather(k_v, [i])
                yout_v[pl.ds(off, _L)] = av + xv * kv

        in_copy(0, 0).start()
        in_copy(1, 1).start()
        for b in range(2):
            in_copy(b, b).wait()
            compute(b)
            out_copy(b, b).start()
            in_copy(b + 2, b).start()

        def pair(p, carry):
            for b in range(2):
                c = 2 * p + b
                in_copy(c, b).wait()
                out_copy(c - 2, b).wait()
                compute(b)
                out_copy(c, b).start()
                in_copy(c + 2, b).start()
            return carry

        lax.fori_loop(1, n_chunks // 2 - 1, pair, 0)

        for b in range(2):
            c = n_chunks - 2 + b
            in_copy(c, b).wait()
            out_copy(c - 2, b).wait()
            compute(b)
            out_copy(c, b).start()
        for b in range(2):
            out_copy(n_chunks - 2 + b, b).wait()

    return k


def kernel(x, mode, Br, Bl, Kl, Kr, Yidx):
    del mode
    orig_shape = x.shape
    xf = x.reshape(-1)
    m = xf.shape[0]

    f32 = jnp.float32
    Br = Br.astype(f32)
    Bl = Bl.astype(f32)
    inv_d = _NBINS / (Br - Bl)
    d = (Br - Bl) / _NBINS

    j = jnp.arange(_NBINS, dtype=f32)
    k_in = (Yidx[1:] - Yidx[:-1]) * inv_d
    b_j = Bl + j * d
    a_in = Yidx[:-1] - b_j * k_in
    k_full = jnp.concatenate(
        [Kl[None].astype(f32), k_in, Kr[None].astype(f32)])
    a_full = jnp.concatenate(
        [(Yidx[0] - Bl * Kl)[None], a_in, (Yidx[-1] - Br * Kr)[None]])
    a_tab = jnp.zeros((32,), f32).at[: _NBINS + 2].set(a_full)
    k_tab = jnp.zeros((32,), f32).at[: _NBINS + 2].set(k_full)

    scale_vec = jnp.full((_L,), inv_d, f32)
    shift_vec = jnp.full((_L,), 1.0 - Bl * inv_d, f32)

    out = _pwla_call(m)(xf, a_tab, k_tab, scale_vec, shift_vec)
    return out.reshape(orig_shape)


# speedup vs baseline: 4.0282x; 1.9375x over previous
import functools

import jax
import jax.numpy as jnp
from jax import lax
from jax.experimental import pallas as pl
from jax.experimental.pallas import tpu as pltpu
from jax.experimental.pallas import tpu_sc as plsc

_NBINS = 16
_NC, _NS, _L = 2, 16, 16
_NW = _NC * _NS
_CROWS = 8
_UNROLL = 8


def _pwla_call(nrows, ncols):
    rows_per_w = nrows // _NW
    n_chunks = rows_per_w // _CROWS
    ch = _CROWS * ncols

    mesh = plsc.VectorSubcoreMesh(
        core_axis_name="c", subcore_axis_name="s",
        num_cores=_NC, num_subcores=_NS)

    @functools.partial(
        pl.kernel,
        out_type=jax.ShapeDtypeStruct((nrows, ncols), jnp.float32),
        mesh=mesh,
        compiler_params=pltpu.CompilerParams(needs_layout_passes=False),
        scratch_types=[
            pltpu.VMEM((_CROWS, ncols), jnp.float32),
            pltpu.VMEM((_CROWS, ncols), jnp.float32),
            pltpu.VMEM((_CROWS, ncols), jnp.float32),
            pltpu.VMEM((_CROWS, ncols), jnp.float32),
            pltpu.VMEM((32,), jnp.float32),
            pltpu.VMEM((32,), jnp.float32),
            pltpu.VMEM((_L,), jnp.float32),
            pltpu.VMEM((_L,), jnp.float32),
            pltpu.SemaphoreType.DMA,
            pltpu.SemaphoreType.DMA,
            pltpu.SemaphoreType.DMA,
            pltpu.SemaphoreType.DMA,
        ],
    )
    def k(x_hbm, a_hbm, k_hbm, sc_hbm, sh_hbm, out_hbm,
          xin0, xin1, yout0, yout1, a_v, k_v, sc_v, sh_v,
          isem0, isem1, osem0, osem1):
        wid = lax.axis_index("s") * _NC + lax.axis_index("c")
        base_row = wid * rows_per_w

        pltpu.sync_copy(a_hbm, a_v)
        pltpu.sync_copy(k_hbm, k_v)
        pltpu.sync_copy(sc_hbm, sc_v)
        pltpu.sync_copy(sh_hbm, sh_v)
        scale = sc_v[...]
        shift = sh_v[...]

        xbufs = (xin0, xin1)
        ybufs = (yout0, yout1)
        isems = (isem0, isem1)
        osems = (osem0, osem1)

        def in_copy(c, b):
            return pltpu.make_async_copy(
                x_hbm.at[pl.ds(base_row + c * _CROWS, _CROWS), :],
                xbufs[b], isems[b])

        def out_copy(c, b):
            return pltpu.make_async_copy(
                ybufs[b],
                out_hbm.at[pl.ds(base_row + c * _CROWS, _CROWS), :],
                osems[b])

        def compute(b):
            xin_v = xbufs[b]
            yout_v = ybufs[b]
            for r in range(_CROWS):

                @plsc.parallel_loop(0, ncols, step=_L, unroll=_UNROLL)
                def vbody(off):
                    xv = xin_v[r, pl.ds(off, _L)]
                    t = jnp.minimum(
                        jnp.maximum(xv * scale + shift, 0.0),
                        float(_NBINS + 1))
                    i = t.astype(jnp.int32)
                    av = plsc.load_gather(a_v, [i])
                    kv = plsc.load_gather(k_v, [i])
                    yout_v[r, pl.ds(off, _L)] = av + xv * kv

        in_copy(0, 0).start()
        in_copy(1, 1).start()
        for b in range(2):
            in_copy(b, b).wait()
            compute(b)
            out_copy(b, b).start()
            in_copy(b + 2, b).start()

        def pair(p, carry):
            for b in range(2):
                c = 2 * p + b
                in_copy(c, b).wait()
                out_copy(c - 2, b).wait()
                compute(b)
                out_copy(c, b).start()
                in_copy(c + 2, b).start()
            return carry

        lax.fori_loop(1, n_chunks // 2 - 1, pair, 0)

        for b in range(2):
            c = n_chunks - 2 + b
            in_copy(c, b).wait()
            out_copy(c - 2, b).wait()
            compute(b)
            out_copy(c, b).start()
        for b in range(2):
            out_copy(n_chunks - 2 + b, b).wait()

    return k


def kernel(x, mode, Br, Bl, Kl, Kr, Yidx):
    del mode
    orig_shape = x.shape
    ncols = x.shape[-1]
    x2 = x.reshape(-1, ncols)
    nrows = x2.shape[0]

    f32 = jnp.float32
    Br = Br.astype(f32)
    Bl = Bl.astype(f32)
    inv_d = _NBINS / (Br - Bl)
    d = (Br - Bl) / _NBINS

    j = jnp.arange(_NBINS, dtype=f32)
    k_in = (Yidx[1:] - Yidx[:-1]) * inv_d
    b_j = Bl + j * d
    a_in = Yidx[:-1] - b_j * k_in
    k_full = jnp.concatenate(
        [Kl[None].astype(f32), k_in, Kr[None].astype(f32)])
    a_full = jnp.concatenate(
        [(Yidx[0] - Bl * Kl)[None], a_in, (Yidx[-1] - Br * Kr)[None]])
    a_tab = jnp.zeros((32,), f32).at[: _NBINS + 2].set(a_full)
    k_tab = jnp.zeros((32,), f32).at[: _NBINS + 2].set(k_full)

    scale_vec = jnp.full((_L,), inv_d, f32)
    shift_vec = jnp.full((_L,), 1.0 - Bl * inv_d, f32)

    out = _pwla_call(nrows, ncols)(x2, a_tab, k_tab, scale_vec, shift_vec)
    return out.reshape(orig_shape)


# speedup vs baseline: 4.8102x; 1.1941x over previous
import functools

import jax
import jax.numpy as jnp
from jax import lax
from jax.experimental import pallas as pl
from jax.experimental.pallas import tpu as pltpu
from jax.experimental.pallas import tpu_sc as plsc

_NBINS = 16
_NC, _NS, _L = 2, 16, 16
_NW = _NC * _NS
_CROWS = 8
_UNROLL = 8


def _pwla_call(nrows, ncols):
    rows_per_w = nrows // _NW
    n_chunks = rows_per_w // _CROWS
    ch = _CROWS * ncols

    mesh = plsc.VectorSubcoreMesh(
        core_axis_name="c", subcore_axis_name="s",
        num_cores=_NC, num_subcores=_NS)

    @functools.partial(
        pl.kernel,
        out_type=jax.ShapeDtypeStruct((nrows, ncols), jnp.float32),
        mesh=mesh,
        compiler_params=pltpu.CompilerParams(needs_layout_passes=False),
        scratch_types=[
            pltpu.VMEM((_CROWS, ncols), jnp.float32),
            pltpu.VMEM((_CROWS, ncols), jnp.float32),
            pltpu.VMEM((_CROWS, ncols), jnp.float32),
            pltpu.VMEM((_CROWS, ncols), jnp.float32),
            pltpu.VMEM((32,), jnp.float32),
            pltpu.VMEM((32,), jnp.float32),
            pltpu.VMEM((_L,), jnp.float32),
            pltpu.VMEM((_L,), jnp.float32),
            pltpu.SemaphoreType.DMA,
            pltpu.SemaphoreType.DMA,
            pltpu.SemaphoreType.DMA,
            pltpu.SemaphoreType.DMA,
        ],
    )
    def k(x_hbm, a_hbm, k_hbm, sc_hbm, sh_hbm, out_hbm,
          xin0, xin1, yout0, yout1, a_v, k_v, sc_v, sh_v,
          isem0, isem1, osem0, osem1):
        wid = lax.axis_index("s") * _NC + lax.axis_index("c")
        base_row = wid * rows_per_w

        xbufs = (xin0, xin1)
        ybufs = (yout0, yout1)
        isems = (isem0, isem1)
        osems = (osem0, osem1)

        def in_copy(c, b):
            return pltpu.make_async_copy(
                x_hbm.at[pl.ds(base_row + c * _CROWS, _CROWS), :],
                xbufs[b], isems[b])

        def out_copy(c, b):
            return pltpu.make_async_copy(
                ybufs[b],
                out_hbm.at[pl.ds(base_row + c * _CROWS, _CROWS), :],
                osems[b])

        col_shift = ncols.bit_length() - 1

        def compute(b, scale, shift):
            xin_v = xbufs[b]
            yout_v = ybufs[b]

            @plsc.parallel_loop(0, ch, step=_L, unroll=_UNROLL)
            def vbody(off):
                r = lax.shift_right_logical(off, col_shift)
                col = lax.bitwise_and(off, ncols - 1)
                xv = xin_v[r, pl.ds(col, _L)]
                t = jnp.minimum(
                    jnp.maximum(xv * scale + shift, 0.0),
                    float(_NBINS + 1))
                i = t.astype(jnp.int32)
                av = plsc.load_gather(a_v, [i])
                kv = plsc.load_gather(k_v, [i])
                yout_v[r, pl.ds(col, _L)] = av + xv * kv

        in_copy(0, 0).start()
        in_copy(1, 1).start()
        pltpu.sync_copy(a_hbm, a_v)
        pltpu.sync_copy(k_hbm, k_v)
        pltpu.sync_copy(sc_hbm, sc_v)
        pltpu.sync_copy(sh_hbm, sh_v)
        scale = sc_v[...]
        shift = sh_v[...]
        compute = functools.partial(compute, scale=scale, shift=shift)
        for b in range(2):
            in_copy(b, b).wait()
            compute(b)
            out_copy(b, b).start()
            in_copy(b + 2, b).start()

        def pair(p, carry):
            for b in range(2):
                c = 2 * p + b
                in_copy(c, b).wait()
                out_copy(c - 2, b).wait()
                compute(b)
                out_copy(c, b).start()
                in_copy(c + 2, b).start()
            return carry

        lax.fori_loop(1, n_chunks // 2 - 1, pair, 0)

        for b in range(2):
            c = n_chunks - 2 + b
            in_copy(c, b).wait()
            out_copy(c - 2, b).wait()
            compute(b)
            out_copy(c, b).start()
        for b in range(2):
            out_copy(n_chunks - 2 + b, b).wait()

    return k


def kernel(x, mode, Br, Bl, Kl, Kr, Yidx):
    del mode
    orig_shape = x.shape
    ncols = x.shape[-1]
    x2 = x.reshape(-1, ncols)
    nrows = x2.shape[0]

    f32 = jnp.float32
    Br = Br.astype(f32)
    Bl = Bl.astype(f32)
    inv_d = _NBINS / (Br - Bl)
    d = (Br - Bl) / _NBINS

    j = jnp.arange(_NBINS, dtype=f32)
    k_in = (Yidx[1:] - Yidx[:-1]) * inv_d
    b_j = Bl + j * d
    a_in = Yidx[:-1] - b_j * k_in
    k_full = jnp.concatenate(
        [Kl[None].astype(f32), k_in, Kr[None].astype(f32)])
    a_full = jnp.concatenate(
        [(Yidx[0] - Bl * Kl)[None], a_in, (Yidx[-1] - Br * Kr)[None]])
    a_tab = jnp.zeros((32,), f32).at[: _NBINS + 2].set(a_full)
    k_tab = jnp.zeros((32,), f32).at[: _NBINS + 2].set(k_full)

    scale_vec = jnp.full((_L,), inv_d, f32)
    shift_vec = jnp.full((_L,), 1.0 - Bl * inv_d, f32)

    out = _pwla_call(nrows, ncols)(x2, a_tab, k_tab, scale_vec, shift_vec)
    return out.reshape(orig_shape)


# speedup vs baseline: 4.8165x; 1.0013x over previous
import functools

import jax
import jax.numpy as jnp
from jax import lax
from jax.experimental import pallas as pl
from jax.experimental.pallas import tpu as pltpu
from jax.experimental.pallas import tpu_sc as plsc

_NBINS = 16
_NC, _NS, _L = 2, 16, 16
_NW = _NC * _NS
_CROWS = 8
_UNROLL = 8


def _pwla_call(nrows, ncols):
    rows_per_w = nrows // _NW
    n_chunks = rows_per_w // _CROWS
    ch = _CROWS * ncols

    mesh = plsc.VectorSubcoreMesh(
        core_axis_name="c", subcore_axis_name="s",
        num_cores=_NC, num_subcores=_NS)

    @functools.partial(
        pl.kernel,
        out_type=jax.ShapeDtypeStruct((nrows, ncols), jnp.float32),
        mesh=mesh,
        compiler_params=pltpu.CompilerParams(
            needs_layout_passes=False, skip_device_barrier=True),
        scratch_types=[
            pltpu.VMEM((_CROWS, ncols), jnp.float32),
            pltpu.VMEM((_CROWS, ncols), jnp.float32),
            pltpu.VMEM((_CROWS, ncols), jnp.float32),
            pltpu.VMEM((_CROWS, ncols), jnp.float32),
            pltpu.VMEM((32,), jnp.float32),
            pltpu.VMEM((32,), jnp.float32),
            pltpu.VMEM((_L,), jnp.float32),
            pltpu.VMEM((_L,), jnp.float32),
            pltpu.SemaphoreType.DMA,
            pltpu.SemaphoreType.DMA,
            pltpu.SemaphoreType.DMA,
            pltpu.SemaphoreType.DMA,
        ],
    )
    def k(x_hbm, a_hbm, k_hbm, sc_hbm, sh_hbm, out_hbm,
          xin0, xin1, yout0, yout1, a_v, k_v, sc_v, sh_v,
          isem0, isem1, osem0, osem1):
        wid = lax.axis_index("s") * _NC + lax.axis_index("c")
        base_row = wid * rows_per_w

        xbufs = (xin0, xin1)
        ybufs = (yout0, yout1)
        isems = (isem0, isem1)
        osems = (osem0, osem1)

        def in_copy(c, b):
            return pltpu.make_async_copy(
                x_hbm.at[pl.ds(base_row + c * _CROWS, _CROWS), :],
                xbufs[b], isems[b])

        def out_copy(c, b):
            return pltpu.make_async_copy(
                ybufs[b],
                out_hbm.at[pl.ds(base_row + c * _CROWS, _CROWS), :],
                osems[b])

        col_shift = ncols.bit_length() - 1

        def compute(b, scale, shift):
            xin_v = xbufs[b]
            yout_v = ybufs[b]

            @plsc.parallel_loop(0, ch, step=_L, unroll=_UNROLL)
            def vbody(off):
                r = lax.shift_right_logical(off, col_shift)
                col = lax.bitwise_and(off, ncols - 1)
                xv = xin_v[r, pl.ds(col, _L)]
                t = jnp.minimum(
                    jnp.maximum(xv * scale + shift, 0.0),
                    float(_NBINS + 1))
                i = t.astype(jnp.int32)
                av = plsc.load_gather(a_v, [i])
                kv = plsc.load_gather(k_v, [i])
                yout_v[r, pl.ds(col, _L)] = av + xv * kv

        in_copy(0, 0).start()
        in_copy(1, 1).start()
        pltpu.sync_copy(a_hbm, a_v)
        pltpu.sync_copy(k_hbm, k_v)
        pltpu.sync_copy(sc_hbm, sc_v)
        pltpu.sync_copy(sh_hbm, sh_v)
        scale = sc_v[...]
        shift = sh_v[...]
        compute = functools.partial(compute, scale=scale, shift=shift)
        for b in range(2):
            in_copy(b, b).wait()
            compute(b)
            out_copy(b, b).start()
            in_copy(b + 2, b).start()

        def pair(p, carry):
            for b in range(2):
                c = 2 * p + b
                in_copy(c, b).wait()
                out_copy(c - 2, b).wait()
                compute(b)
                out_copy(c, b).start()
                in_copy(c + 2, b).start()
            return carry

        lax.fori_loop(1, n_chunks // 2 - 1, pair, 0)

        for b in range(2):
            c = n_chunks - 2 + b
            in_copy(c, b).wait()
            out_copy(c - 2, b).wait()
            compute(b)
            out_copy(c, b).start()
        for b in range(2):
            out_copy(n_chunks - 2 + b, b).wait()

    return k


def kernel(x, mode, Br, Bl, Kl, Kr, Yidx):
    del mode
    orig_shape = x.shape
    ncols = x.shape[-1]
    x2 = x.reshape(-1, ncols)
    nrows = x2.shape[0]

    f32 = jnp.float32
    Br = Br.astype(f32)
    Bl = Bl.astype(f32)
    inv_d = _NBINS / (Br - Bl)
    d = (Br - Bl) / _NBINS

    j = jnp.arange(_NBINS, dtype=f32)
    k_in = (Yidx[1:] - Yidx[:-1]) * inv_d
    b_j = Bl + j * d
    a_in = Yidx[:-1] - b_j * k_in
    k_full = jnp.concatenate(
        [Kl[None].astype(f32), k_in, Kr[None].astype(f32)])
    a_full = jnp.concatenate(
        [(Yidx[0] - Bl * Kl)[None], a_in, (Yidx[-1] - Br * Kr)[None]])
    a_tab = jnp.zeros((32,), f32).at[: _NBINS + 2].set(a_full)
    k_tab = jnp.zeros((32,), f32).at[: _NBINS + 2].set(k_full)

    scale_vec = jnp.full((_L,), inv_d, f32)
    shift_vec = jnp.full((_L,), 1.0 - Bl * inv_d, f32)

    out = _pwla_call(nrows, ncols)(x2, a_tab, k_tab, scale_vec, shift_vec)
    return out.reshape(orig_shape)
